# baseline (speedup 1.0000x reference)
import functools
import math

import jax
import jax.numpy as jnp
from jax import lax
from jax.experimental import pallas as pl
from jax.experimental.pallas import tpu as pltpu
from jax.experimental.pallas import tpu_sc as plsc

_L = 16

_ATAN_C = (
    0.99999612,
    -0.33317369,
    0.19807802,
    -0.13233266,
    0.07962221,
    -0.03360298,
    0.0068114,
)
_HALF_PI = math.pi / 2


def _atan(x):
    w = jnp.abs(x)
    inv = 1.0 / w
    t = jnp.minimum(w, inv)
    z = t * t
    p = jnp.float32(_ATAN_C[-1])
    for c in _ATAN_C[-2::-1]:
        p = p * z + jnp.float32(c)
    p = p * t
    r = jnp.where(w > 1.0, _HALF_PI - p, p)
    return jnp.where(x < 0.0, -r, r)


def _make_sc_kernel(n, num_classes):
    info = plsc.get_sparse_core_info()
    nc, ns = info.num_cores, info.num_subcores
    nw = nc * ns
    assert n % (nw * _L) == 0
    bpw = n // nw
    groups = bpw // _L
    chunk = 128
    nchunks = bpw // chunk

    mesh = plsc.VectorSubcoreMesh(core_axis_name="c", subcore_axis_name="s")

    @functools.partial(
        pl.kernel,
        mesh=mesh,
        compiler_params=pltpu.CompilerParams(
            needs_layout_passes=False, use_tc_tiling_on_sc=False),
        out_type=[
            jax.ShapeDtypeStruct((n,), jnp.float32),
            jax.ShapeDtypeStruct((n, 3), jnp.float32),
            jax.ShapeDtypeStruct((n, 8), jnp.float32),
            jax.ShapeDtypeStruct((n,), jnp.float32),
        ],
        scratch_types=[
            pltpu.VMEM((bpw,), jnp.int32),
            pltpu.VMEM((nchunks, chunk), jnp.int32),
            pltpu.VMEM((3, nchunks, chunk), jnp.int32),
            pltpu.VMEM((bpw,), jnp.float32),
            pltpu.VMEM((3, bpw), jnp.float32),
            pltpu.VMEM((bpw, 3), jnp.float32),
            pltpu.VMEM((bpw, 8), jnp.float32),
            pltpu.VMEM((bpw,), jnp.float32),
            pltpu.VMEM((bpw,), jnp.float32),
            pltpu.SemaphoreType.DMA,
        ],
    )
    def sc_kernel(dep_hbm, dims_hbm, rots_hbm, lab_hbm,
                  d_out, dims_out, rots_out, alpha_out,
                  lab_v, idx_v, idx3_v, dep_v, dcol_v, dims_v, rots_v,
                  dout_v, alph_v, sem):
        wid = lax.axis_index("s") * nc + lax.axis_index("c")
        base = wid * bpw
        pltpu.sync_copy(lab_hbm.at[pl.ds(base, bpw)], lab_v)

        iota = lax.iota(jnp.int32, _L)

        per_chunk = chunk // _L

        def build_idx(g, _):
            off = g * _L
            c = g >> 3 if per_chunk == 8 else g // per_chunk
            k = off - c * chunk
            lab = lab_v[pl.ds(off, _L)]
            row = (base + off) + iota
            gi = row * num_classes + lab
            idx_v[c, pl.ds(k, _L)] = gi
            gi3 = gi * 3
            idx3_v[0, c, pl.ds(k, _L)] = gi3
            idx3_v[1, c, pl.ds(k, _L)] = gi3 + 1
            idx3_v[2, c, pl.ds(k, _L)] = gi3 + 2
            return 0

        lax.fori_loop(0, groups, build_idx, 0)

        copies = []
        for c in range(nchunks):
            o = c * chunk
            copies.append(pltpu.async_copy(
                dep_hbm.at[idx_v.at[c]], dep_v.at[pl.ds(o, chunk)], sem))
            copies.append(pltpu.async_copy(
                rots_hbm.at[idx_v.at[c]], rots_v.at[pl.ds(o, chunk)], sem))
            for j in range(3):
                copies.append(pltpu.async_copy(
                    dims_hbm.at[idx3_v.at[j].at[c]],
                    dcol_v.at[j].at[pl.ds(o, chunk)], sem))
        for cp in copies:
            cp.wait()

        def compute(g, _):
            off = g * _L
            dep = dep_v[pl.ds(off, _L)]
            dout_v[pl.ds(off, _L)] = jnp.exp(dep) - 1.0

            rows = off + iota
            for j in range(3):
                plsc.store_scatter(
                    dims_v, [rows, jnp.full((_L,), j, jnp.int32)],
                    dcol_v[j, pl.ds(off, _L)])

            def col(j):
                cj = jnp.full((_L,), j, jnp.int32)
                return plsc.load_gather(rots_v, [rows, cj])
            r1, r2, r3 = col(1), col(2), col(3)
            r5, r6, r7 = col(5), col(6), col(7)
            a1 = _atan(r2 / r3) - _HALF_PI
            a2 = _atan(r6 / r7) + _HALF_PI
            alph_v[pl.ds(off, _L)] = jnp.where(r1 > r5, a1, a2)
            return 0

        lax.fori_loop(0, groups, compute, 0)

        pltpu.sync_copy(dout_v, d_out.at[pl.ds(base, bpw)])
        pltpu.sync_copy(dims_v, dims_out.at[pl.ds(base, bpw)])
        pltpu.sync_copy(rots_v, rots_out.at[pl.ds(base, bpw)])
        pltpu.sync_copy(alph_v, alpha_out.at[pl.ds(base, bpw)])

    return sc_kernel


def kernel(depths, dims, rots, labels):
    n, num_classes = depths.shape
    dep_flat = depths.reshape(n * num_classes)
    dims_flat = dims.reshape(n * num_classes * 3)
    rots_flat = rots.reshape(n * num_classes, 8)
    lab = labels.astype(jnp.int32)
    d, dims_g, rots_g, alphas = _make_sc_kernel(n, num_classes)(
        dep_flat, dims_flat, rots_flat, lab)
    return d.reshape(n, 1), dims_g, rots_g, alphas


# speedup vs baseline: 2.8697x; 2.8697x over previous
import functools
import math

import jax
import jax.numpy as jnp
from jax import lax
from jax.experimental import pallas as pl
from jax.experimental.pallas import tpu as pltpu
from jax.experimental.pallas import tpu_sc as plsc

_L = 16

_ATAN_C = (
    0.99999612,
    -0.33317369,
    0.19807802,
    -0.13233266,
    0.07962221,
    -0.03360298,
    0.0068114,
)
_HALF_PI = math.pi / 2


def _atan(x):
    w = jnp.abs(x)
    inv = 1.0 / w
    t = jnp.minimum(w, inv)
    z = t * t
    p = jnp.float32(_ATAN_C[-1])
    for c in _ATAN_C[-2::-1]:
        p = p * z + jnp.float32(c)
    p = p * t
    r = jnp.where(w > 1.0, _HALF_PI - p, p)
    return jnp.where(x < 0.0, -r, r)


def _make_sc_kernel(n, num_classes):
    info = plsc.get_sparse_core_info()
    nc, ns = info.num_cores, info.num_subcores
    nw = nc * ns
    assert n % (nw * _L) == 0
    bpw = n // nw
    groups = bpw // _L
    chunk = 128
    nchunks = bpw // chunk
    per_chunk = chunk // _L

    mesh = plsc.VectorSubcoreMesh(core_axis_name="c", subcore_axis_name="s")

    @functools.partial(
        pl.kernel,
        mesh=mesh,
        compiler_params=pltpu.CompilerParams(
            needs_layout_passes=False, use_tc_tiling_on_sc=False),
        out_type=[
            jax.ShapeDtypeStruct((n,), jnp.float32),
            jax.ShapeDtypeStruct((3, n), jnp.float32),
            jax.ShapeDtypeStruct((8, n), jnp.float32),
            jax.ShapeDtypeStruct((n,), jnp.float32),
        ],
        scratch_types=[
            pltpu.VMEM((bpw,), jnp.int32),
            pltpu.VMEM((nchunks, chunk), jnp.int32),
            pltpu.VMEM((nchunks, chunk), jnp.int32),
            pltpu.VMEM((3, nchunks, chunk), jnp.int32),
            pltpu.VMEM((bpw,), jnp.float32),
            pltpu.VMEM((3, bpw), jnp.float32),
            pltpu.VMEM((bpw, 8), jnp.float32),
            pltpu.VMEM((8, bpw), jnp.float32),
            pltpu.VMEM((bpw,), jnp.float32),
            pltpu.VMEM((bpw,), jnp.float32),
            pltpu.SemaphoreType.DMA,
        ],
    )
    def sc_kernel(dep_hbm, dims_hbm, rots_hbm, lab_hbm,
                  d_out, dims_out, rots_out, alpha_out,
                  lab_v, idxr_v, idxd_v, idx3_v, dep_v, dcol_v, rots_v,
                  rcomp_v, dout_v, alph_v, sem):
        wid = lax.axis_index("s") * nc + lax.axis_index("c")
        base = wid * bpw
        pltpu.sync_copy(lab_hbm.at[pl.ds(base, bpw)], lab_v)

        iota = lax.iota(jnp.int32, _L)

        def build_idx(g, _):
            off = g * _L
            c = g // per_chunk
            k = off - c * chunk
            lab = lab_v[pl.ds(off, _L)]
            row = (base + off) + iota
            idxr_v[c, pl.ds(k, _L)] = row * num_classes + lab
            idxd_v[c, pl.ds(k, _L)] = lab * n + row
            g3 = lab * (3 * n) + row
            idx3_v[0, c, pl.ds(k, _L)] = g3
            idx3_v[1, c, pl.ds(k, _L)] = g3 + n
            idx3_v[2, c, pl.ds(k, _L)] = g3 + 2 * n
            return 0

        lax.fori_loop(0, groups, build_idx, 0)

        copies = []
        for c in range(nchunks):
            o = c * chunk
            copies.append(pltpu.async_copy(
                dep_hbm.at[idxd_v.at[c]], dep_v.at[pl.ds(o, chunk)], sem))
            copies.append(pltpu.async_copy(
                rots_hbm.at[idxr_v.at[c]], rots_v.at[pl.ds(o, chunk)], sem))
            for j in range(3):
                copies.append(pltpu.async_copy(
                    dims_hbm.at[idx3_v.at[j].at[c]],
                    dcol_v.at[j].at[pl.ds(o, chunk)], sem))
        for cp in copies:
            cp.wait()

        def compute(g, _):
            off = g * _L
            dep = dep_v[pl.ds(off, _L)]
            dout_v[pl.ds(off, _L)] = jnp.exp(dep) - 1.0

            rows = off + iota
            r = []
            for j in range(8):
                cj = jnp.full((_L,), j, jnp.int32)
                v = plsc.load_gather(rots_v, [rows, cj])
                rcomp_v[j, pl.ds(off, _L)] = v
                r.append(v)
            a1 = _atan(r[2] / r[3]) - _HALF_PI
            a2 = _atan(r[6] / r[7]) + _HALF_PI
            alph_v[pl.ds(off, _L)] = jnp.where(r[1] > r[5], a1, a2)
            return 0

        lax.fori_loop(0, groups, compute, 0)

        pltpu.sync_copy(dout_v, d_out.at[pl.ds(base, bpw)])
        pltpu.sync_copy(alph_v, alpha_out.at[pl.ds(base, bpw)])
        for j in range(3):
            pltpu.sync_copy(dcol_v.at[j], dims_out.at[j].at[pl.ds(base, bpw)])
        for j in range(8):
            pltpu.sync_copy(rcomp_v.at[j], rots_out.at[j].at[pl.ds(base, bpw)])

    return sc_kernel


def kernel(depths, dims, rots, labels):
    n, num_classes = depths.shape
    dep_t = depths.T.reshape(n * num_classes)
    dims_t = dims.T.reshape(n * num_classes * 3)
    rots_flat = rots.reshape(n * num_classes, 8)
    lab = labels.astype(jnp.int32)
    d, dims_t3, rots_t8, alphas = _make_sc_kernel(n, num_classes)(
        dep_t, dims_t, rots_flat, lab)
    return d.reshape(n, 1), dims_t3.T, rots_t8.T, alphas


# speedup vs baseline: 3.6998x; 1.2893x over previous
import functools
import math

import jax
import jax.numpy as jnp
from jax import lax
from jax.experimental import pallas as pl
from jax.experimental.pallas import tpu as pltpu
from jax.experimental.pallas import tpu_sc as plsc

_L = 16

_ATAN_C = (
    0.99999612,
    -0.33317369,
    0.19807802,
    -0.13233266,
    0.07962221,
    -0.03360298,
    0.0068114,
)
_HALF_PI = math.pi / 2


def _atan(x):
    w = jnp.abs(x)
    inv = 1.0 / w
    t = jnp.minimum(w, inv)
    z = t * t
    p = jnp.float32(_ATAN_C[-1])
    for c in _ATAN_C[-2::-1]:
        p = p * z + jnp.float32(c)
    p = p * t
    r = jnp.where(w > 1.0, _HALF_PI - p, p)
    return jnp.where(x < 0.0, -r, r)


def _make_sc_kernel(n, num_classes):
    info = plsc.get_sparse_core_info()
    nc, ns = info.num_cores, info.num_subcores
    nw = nc * ns
    assert n % (nw * _L) == 0
    bpw = n // nw
    groups = bpw // _L
    chunk = 128
    nchunks = bpw // chunk
    per_chunk = chunk // _L

    mesh = plsc.VectorSubcoreMesh(core_axis_name="c", subcore_axis_name="s")

    @functools.partial(
        pl.kernel,
        mesh=mesh,
        compiler_params=pltpu.CompilerParams(
            needs_layout_passes=False, use_tc_tiling_on_sc=False),
        out_type=[
            jax.ShapeDtypeStruct((n,), jnp.float32),
            jax.ShapeDtypeStruct((3, n), jnp.float32),
            jax.ShapeDtypeStruct((8, n), jnp.float32),
            jax.ShapeDtypeStruct((n,), jnp.float32),
        ],
        scratch_types=[
            pltpu.VMEM((bpw,), jnp.int32),
            pltpu.VMEM((nchunks, chunk), jnp.int32),
            pltpu.VMEM((nchunks, chunk), jnp.int32),
            pltpu.VMEM((3, nchunks, chunk), jnp.int32),
            pltpu.VMEM((bpw,), jnp.float32),
            pltpu.VMEM((3, bpw), jnp.float32),
            pltpu.VMEM((bpw, 8), jnp.float32),
            pltpu.VMEM((8, bpw), jnp.float32),
            pltpu.VMEM((bpw,), jnp.float32),
            pltpu.VMEM((bpw,), jnp.float32),
            pltpu.SemaphoreType.DMA,
        ],
    )
    def sc_kernel(dep_hbm, dims_hbm, rots_hbm, lab_hbm,
                  d_out, dims_out, rots_out, alpha_out,
                  lab_v, idxr_v, idxd_v, idx3_v, dep_v, dcol_v, rots_v,
                  rcomp_v, dout_v, alph_v, sem):
        wid = lax.axis_index("s") * nc + lax.axis_index("c")
        base = wid * bpw
        pltpu.sync_copy(lab_hbm.at[pl.ds(base, bpw)], lab_v)

        iota = lax.iota(jnp.int32, _L)

        def build_idx(g, _):
            off = g * _L
            c = g // per_chunk
            k = off - c * chunk
            lab = lab_v[pl.ds(off, _L)]
            row = (base + off) + iota
            idxr_v[c, pl.ds(k, _L)] = row * num_classes + lab
            tile = (row >> 7) * 1024 + (row & 127)
            idxd_v[c, pl.ds(k, _L)] = (
                (lab >> 3) * (8 * n) + ((lab & 7) << 7) + tile)
            cd = lab * 3
            for j in range(3):
                cdj = cd + j
                idx3_v[j, c, pl.ds(k, _L)] = (
                    (cdj >> 3) * (8 * n) + ((cdj & 7) << 7) + tile)
            return 0

        lax.fori_loop(0, groups, build_idx, 0)

        copies = []
        for c in range(nchunks):
            o = c * chunk
            copies.append(pltpu.async_copy(
                dep_hbm.at[idxd_v.at[c]], dep_v.at[pl.ds(o, chunk)], sem))
            copies.append(pltpu.async_copy(
                rots_hbm.at[idxr_v.at[c]], rots_v.at[pl.ds(o, chunk)], sem))
            for j in range(3):
                copies.append(pltpu.async_copy(
                    dims_hbm.at[idx3_v.at[j].at[c]],
                    dcol_v.at[j].at[pl.ds(o, chunk)], sem))
        for cp in copies:
            cp.wait()

        def compute(g, _):
            off = g * _L
            dep = dep_v[pl.ds(off, _L)]
            dout_v[pl.ds(off, _L)] = jnp.exp(dep) - 1.0

            rows = off + iota
            r = []
            for j in range(8):
                cj = jnp.full((_L,), j, jnp.int32)
                v = plsc.load_gather(rots_v, [rows, cj])
                rcomp_v[j, pl.ds(off, _L)] = v
                r.append(v)
            a1 = _atan(r[2] / r[3]) - _HALF_PI
            a2 = _atan(r[6] / r[7]) + _HALF_PI
            alph_v[pl.ds(off, _L)] = jnp.where(r[1] > r[5], a1, a2)
            return 0

        lax.fori_loop(0, groups, compute, 0)

        pltpu.sync_copy(dout_v, d_out.at[pl.ds(base, bpw)])
        pltpu.sync_copy(alph_v, alpha_out.at[pl.ds(base, bpw)])
        for j in range(3):
            pltpu.sync_copy(dcol_v.at[j], dims_out.at[j].at[pl.ds(base, bpw)])
        for j in range(8):
            pltpu.sync_copy(rcomp_v.at[j], rots_out.at[j].at[pl.ds(base, bpw)])

    return sc_kernel


def kernel(depths, dims, rots, labels):
    n, num_classes = depths.shape
    dep_t = (depths.T.reshape(num_classes // 8, 8, n // 128, 128)
             .transpose(0, 2, 1, 3).reshape(n * num_classes))
    dims_t = (dims.T.reshape(num_classes * 3 // 8, 8, n // 128, 128)
              .transpose(0, 2, 1, 3).reshape(n * num_classes * 3))
    rots_flat = rots.reshape(n * num_classes, 8)
    lab = labels.astype(jnp.int32)
    d, dims_t3, rots_t8, alphas = _make_sc_kernel(n, num_classes)(
        dep_t, dims_t, rots_flat, lab)
    return d.reshape(n, 1), dims_t3.T, rots_t8.T, alphas


# speedup vs baseline: 3.9081x; 1.0563x over previous
import functools
import math

import jax
import jax.numpy as jnp
from jax import lax
from jax.experimental import pallas as pl
from jax.experimental.pallas import tpu as pltpu
from jax.experimental.pallas import tpu_sc as plsc

_L = 16

_ATAN_C = (
    0.99999612,
    -0.33317369,
    0.19807802,
    -0.13233266,
    0.07962221,
    -0.03360298,
    0.0068114,
)
_HALF_PI = math.pi / 2


def _atan(x):
    w = jnp.abs(x)
    inv = 1.0 / w
    t = jnp.minimum(w, inv)
    z = t * t
    p = jnp.float32(_ATAN_C[-1])
    for c in _ATAN_C[-2::-1]:
        p = p * z + jnp.float32(c)
    p = p * t
    r = jnp.where(w > 1.0, _HALF_PI - p, p)
    return jnp.where(x < 0.0, -r, r)


def _make_sc_kernel(n, num_classes):
    info = plsc.get_sparse_core_info()
    nc, ns = info.num_cores, info.num_subcores
    nw = nc * ns
    assert n % (nw * _L) == 0
    bpw = n // nw
    groups = bpw // _L
    chunk = 128
    nchunks = bpw // chunk
    per_chunk = chunk // _L

    mesh = plsc.VectorSubcoreMesh(core_axis_name="c", subcore_axis_name="s")

    @functools.partial(
        pl.kernel,
        mesh=mesh,
        compiler_params=pltpu.CompilerParams(
            needs_layout_passes=False, use_tc_tiling_on_sc=False),
        out_type=[
            jax.ShapeDtypeStruct((n,), jnp.float32),
            jax.ShapeDtypeStruct((3, n), jnp.float32),
            jax.ShapeDtypeStruct((8, n), jnp.float32),
            jax.ShapeDtypeStruct((n,), jnp.float32),
        ],
        scratch_types=[
            pltpu.VMEM((bpw,), jnp.int32),
            pltpu.VMEM((nchunks, chunk), jnp.int32),
            pltpu.VMEM((nchunks, chunk), jnp.int32),
            pltpu.VMEM((3, nchunks, chunk), jnp.int32),
            pltpu.VMEM((bpw,), jnp.float32),
            pltpu.VMEM((3, bpw), jnp.float32),
            pltpu.VMEM((bpw, 8), jnp.float32),
            pltpu.VMEM((8, bpw), jnp.float32),
            pltpu.VMEM((bpw,), jnp.float32),
            pltpu.VMEM((bpw,), jnp.float32),
            pltpu.SemaphoreType.DMA((bpw // 128,)),
            pltpu.SemaphoreType.DMA,
        ],
    )
    def sc_kernel(dep_hbm, dims_hbm, rots_hbm, lab_hbm,
                  d_out, dims_out, rots_out, alpha_out,
                  lab_v, idxr_v, idxd_v, idx3_v, dep_v, dcol_v, rots_v,
                  rcomp_v, dout_v, alph_v, gsem, osem):
        wid = lax.axis_index("s") * nc + lax.axis_index("c")
        base = wid * bpw
        pltpu.sync_copy(lab_hbm.at[pl.ds(base, bpw)], lab_v)

        iota = lax.iota(jnp.int32, _L)

        def build_idx(g, _):
            off = g * _L
            c = g // per_chunk
            k = off - c * chunk
            lab = lab_v[pl.ds(off, _L)]
            row = (base + off) + iota
            idxr_v[c, pl.ds(k, _L)] = row * num_classes + lab
            tile = (row >> 7) * 1024 + (row & 127)
            idxd_v[c, pl.ds(k, _L)] = (
                (lab >> 3) * (8 * n) + ((lab & 7) << 7) + tile)
            cd = lab * 3
            for j in range(3):
                cdj = cd + j
                idx3_v[j, c, pl.ds(k, _L)] = (
                    (cdj >> 3) * (8 * n) + ((cdj & 7) << 7) + tile)
            return 0

        copies = []
        for c in range(nchunks):
            lax.fori_loop(c * per_chunk, (c + 1) * per_chunk, build_idx, 0)
            o = c * chunk
            sem = gsem.at[c]
            copies.append(pltpu.async_copy(
                dep_hbm.at[idxd_v.at[c]], dep_v.at[pl.ds(o, chunk)], sem))
            copies.append(pltpu.async_copy(
                rots_hbm.at[idxr_v.at[c]], rots_v.at[pl.ds(o, chunk)], sem))
            for j in range(3):
                copies.append(pltpu.async_copy(
                    dims_hbm.at[idx3_v.at[j].at[c]],
                    dcol_v.at[j].at[pl.ds(o, chunk)], sem))

        def compute(g, _):
            off = g * _L
            dep = dep_v[pl.ds(off, _L)]
            dout_v[pl.ds(off, _L)] = jnp.exp(dep) - 1.0

            rows = off + iota
            r = []
            for j in range(8):
                cj = jnp.full((_L,), j, jnp.int32)
                v = plsc.load_gather(rots_v, [rows, cj])
                rcomp_v[j, pl.ds(off, _L)] = v
                r.append(v)
            a1 = _atan(r[2] / r[3]) - _HALF_PI
            a2 = _atan(r[6] / r[7]) + _HALF_PI
            alph_v[pl.ds(off, _L)] = jnp.where(r[1] > r[5], a1, a2)
            return 0

        for c in range(nchunks):
            for cp in copies[5 * c:5 * c + 5]:
                cp.wait()
            lax.fori_loop(c * per_chunk, (c + 1) * per_chunk, compute, 0)

        out_copies = [
            pltpu.async_copy(dout_v, d_out.at[pl.ds(base, bpw)], osem),
            pltpu.async_copy(alph_v, alpha_out.at[pl.ds(base, bpw)], osem),
        ]
        for j in range(3):
            out_copies.append(pltpu.async_copy(
                dcol_v.at[j], dims_out.at[j].at[pl.ds(base, bpw)], osem))
        for j in range(8):
            out_copies.append(pltpu.async_copy(
                rcomp_v.at[j], rots_out.at[j].at[pl.ds(base, bpw)], osem))
        for cp in out_copies:
            cp.wait()

    return sc_kernel


def kernel(depths, dims, rots, labels):
    n, num_classes = depths.shape
    dep_t = (depths.T.reshape(num_classes // 8, 8, n // 128, 128)
             .transpose(0, 2, 1, 3).reshape(n * num_classes))
    dims_t = (dims.T.reshape(num_classes * 3 // 8, 8, n // 128, 128)
              .transpose(0, 2, 1, 3).reshape(n * num_classes * 3))
    rots_flat = rots.reshape(n * num_classes, 8)
    lab = labels.astype(jnp.int32)
    d, dims_t3, rots_t8, alphas = _make_sc_kernel(n, num_classes)(
        dep_t, dims_t, rots_flat, lab)
    return d.reshape(n, 1), dims_t3.T, rots_t8.T, alphas


# speedup vs baseline: 3.9221x; 1.0036x over previous
import functools
import math

import jax
import jax.numpy as jnp
from jax import lax
from jax.experimental import pallas as pl
from jax.experimental.pallas import tpu as pltpu
from jax.experimental.pallas import tpu_sc as plsc

_L = 16

_ATAN_C = (
    0.99999612,
    -0.33317369,
    0.19807802,
    -0.13233266,
    0.07962221,
    -0.03360298,
    0.0068114,
)
_HALF_PI = math.pi / 2


def _atan_ratio(num, den):
    an = jnp.abs(num)
    ad = jnp.abs(den)
    t = jnp.minimum(an, ad) / jnp.maximum(an, ad)
    z = t * t
    p = jnp.float32(_ATAN_C[-1])
    for c in _ATAN_C[-2::-1]:
        p = p * z + jnp.float32(c)
    p = p * t
    r = jnp.where(an > ad, _HALF_PI - p, p)
    return jnp.where(num * den < 0.0, -r, r)


def _make_sc_kernel(n, num_classes):
    info = plsc.get_sparse_core_info()
    nc, ns = info.num_cores, info.num_subcores
    nw = nc * ns
    assert n % (nw * _L) == 0
    bpw = n // nw
    groups = bpw // _L
    chunk = 128
    nchunks = bpw // chunk
    per_chunk = chunk // _L

    mesh = plsc.VectorSubcoreMesh(core_axis_name="c", subcore_axis_name="s")

    @functools.partial(
        pl.kernel,
        mesh=mesh,
        compiler_params=pltpu.CompilerParams(
            needs_layout_passes=False, use_tc_tiling_on_sc=False),
        out_type=[
            jax.ShapeDtypeStruct((n,), jnp.float32),
            jax.ShapeDtypeStruct((3, n), jnp.float32),
            jax.ShapeDtypeStruct((8, n), jnp.float32),
            jax.ShapeDtypeStruct((n,), jnp.float32),
        ],
        scratch_types=[
            pltpu.VMEM((bpw,), jnp.int32),
            pltpu.VMEM((nchunks, chunk), jnp.int32),
            pltpu.VMEM((nchunks, chunk), jnp.int32),
            pltpu.VMEM((3, nchunks, chunk), jnp.int32),
            pltpu.VMEM((bpw,), jnp.float32),
            pltpu.VMEM((3, bpw), jnp.float32),
            pltpu.VMEM((bpw, 8), jnp.float32),
            pltpu.VMEM((8, bpw), jnp.float32),
            pltpu.VMEM((bpw,), jnp.float32),
            pltpu.VMEM((bpw,), jnp.float32),
            pltpu.SemaphoreType.DMA((bpw // 128,)),
            pltpu.SemaphoreType.DMA,
        ],
    )
    def sc_kernel(dep_hbm, dims_hbm, rots_hbm, lab_hbm,
                  d_out, dims_out, rots_out, alpha_out,
                  lab_v, idxr_v, idxd_v, idx3_v, dep_v, dcol_v, rots_v,
                  rcomp_v, dout_v, alph_v, gsem, osem):
        wid = lax.axis_index("s") * nc + lax.axis_index("c")
        base = wid * bpw
        pltpu.sync_copy(lab_hbm.at[pl.ds(base, bpw)], lab_v)

        iota = lax.iota(jnp.int32, _L)

        def build_idx(g):
            off = g * _L
            c = g // per_chunk
            k = off - c * chunk
            lab = lab_v[pl.ds(off, _L)]
            row = (base + off) + iota
            idxr_v[c, pl.ds(k, _L)] = row * num_classes + lab
            tile = (row >> 7) * 1024 + (row & 127)
            idxd_v[c, pl.ds(k, _L)] = (
                (lab >> 3) * (8 * n) + ((lab & 7) << 7) + tile)
            cd = lab * 3
            for j in range(3):
                cdj = cd + j
                idx3_v[j, c, pl.ds(k, _L)] = (
                    (cdj >> 3) * (8 * n) + ((cdj & 7) << 7) + tile)

        copies = []
        for c in range(nchunks):
            plsc.parallel_loop(
                c * per_chunk, (c + 1) * per_chunk, unroll=2)(build_idx)
            o = c * chunk
            sem = gsem.at[c]
            copies.append(pltpu.async_copy(
                dep_hbm.at[idxd_v.at[c]], dep_v.at[pl.ds(o, chunk)], sem))
            copies.append(pltpu.async_copy(
                rots_hbm.at[idxr_v.at[c]], rots_v.at[pl.ds(o, chunk)], sem))
            for j in range(3):
                copies.append(pltpu.async_copy(
                    dims_hbm.at[idx3_v.at[j].at[c]],
                    dcol_v.at[j].at[pl.ds(o, chunk)], sem))

        def compute(g):
            off = g * _L
            dep = dep_v[pl.ds(off, _L)]
            dout_v[pl.ds(off, _L)] = jnp.exp(dep) - 1.0

            rows = off + iota
            r = []
            for j in range(8):
                cj = jnp.full((_L,), j, jnp.int32)
                v = plsc.load_gather(rots_v, [rows, cj])
                rcomp_v[j, pl.ds(off, _L)] = v
                r.append(v)
            a1 = _atan_ratio(r[2], r[3]) - _HALF_PI
            a2 = _atan_ratio(r[6], r[7]) + _HALF_PI
            alph_v[pl.ds(off, _L)] = jnp.where(r[1] > r[5], a1, a2)

        for c in range(nchunks):
            for cp in copies[5 * c:5 * c + 5]:
                cp.wait()
            plsc.parallel_loop(
                c * per_chunk, (c + 1) * per_chunk, unroll=2)(compute)

        out_copies = [
            pltpu.async_copy(dout_v, d_out.at[pl.ds(base, bpw)], osem),
            pltpu.async_copy(alph_v, alpha_out.at[pl.ds(base, bpw)], osem),
        ]
        for j in range(3):
            out_copies.append(pltpu.async_copy(
                dcol_v.at[j], dims_out.at[j].at[pl.ds(base, bpw)], osem))
        for j in range(8):
            out_copies.append(pltpu.async_copy(
                rcomp_v.at[j], rots_out.at[j].at[pl.ds(base, bpw)], osem))
        for cp in out_copies:
            cp.wait()

    return sc_kernel


def kernel(depths, dims, rots, labels):
    n, num_classes = depths.shape
    dep_t = (depths.T.reshape(num_classes // 8, 8, n // 128, 128)
             .transpose(0, 2, 1, 3).reshape(n * num_classes))
    dims_t = (dims.T.reshape(num_classes * 3 // 8, 8, n // 128, 128)
              .transpose(0, 2, 1, 3).reshape(n * num_classes * 3))
    rots_flat = rots.reshape(n * num_classes, 8)
    lab = labels.astype(jnp.int32)
    d, dims_t3, rots_t8, alphas = _make_sc_kernel(n, num_classes)(
        dep_t, dims_t, rots_flat, lab)
    return d.reshape(n, 1), dims_t3.T, rots_t8.T, alphas


# speedup vs baseline: 4.2577x; 1.0856x over previous
import functools
import math

import jax
import jax.numpy as jnp
from jax import lax
from jax.experimental import pallas as pl
from jax.experimental.pallas import tpu as pltpu
from jax.experimental.pallas import tpu_sc as plsc

_L = 16

_ATAN_C = (
    0.99999612,
    -0.33317369,
    0.19807802,
    -0.13233266,
    0.07962221,
    -0.03360298,
    0.0068114,
)
_HALF_PI = math.pi / 2


def _atan_ratio(num, den):
    an = jnp.abs(num)
    ad = jnp.abs(den)
    t = jnp.minimum(an, ad) / jnp.maximum(an, ad)
    z = t * t
    p = jnp.float32(_ATAN_C[-1])
    for c in _ATAN_C[-2::-1]:
        p = p * z + jnp.float32(c)
    p = p * t
    r = jnp.where(an > ad, _HALF_PI - p, p)
    return jnp.where(num * den < 0.0, -r, r)


def _make_sc_kernel(n, num_classes):
    info = plsc.get_sparse_core_info()
    nc, ns = info.num_cores, info.num_subcores
    nw = nc * ns
    assert n % (nw * _L) == 0
    bpw = n // nw
    groups = bpw // _L
    chunk = 128
    nchunks = bpw // chunk
    per_chunk = chunk // _L

    mesh = plsc.VectorSubcoreMesh(core_axis_name="c", subcore_axis_name="s")

    @functools.partial(
        pl.kernel,
        mesh=mesh,
        compiler_params=pltpu.CompilerParams(
            needs_layout_passes=False, use_tc_tiling_on_sc=False),
        out_type=[
            jax.ShapeDtypeStruct((n,), jnp.float32),
            jax.ShapeDtypeStruct((n // 128, 4, 128), jnp.float32),
            jax.ShapeDtypeStruct((n // 128, 8, 128), jnp.float32),
            jax.ShapeDtypeStruct((n,), jnp.float32),
        ],
        scratch_types=[
            pltpu.VMEM((bpw,), jnp.int32),
            pltpu.VMEM((nchunks, chunk), jnp.int32),
            pltpu.VMEM((nchunks, chunk), jnp.int32),
            pltpu.VMEM((3, nchunks, chunk), jnp.int32),
            pltpu.VMEM((bpw,), jnp.float32),
            pltpu.VMEM((bpw // 128, 4, 128), jnp.float32),
            pltpu.VMEM((bpw, 8), jnp.float32),
            pltpu.VMEM((bpw // 128, 8, 128), jnp.float32),
            pltpu.VMEM((bpw,), jnp.float32),
            pltpu.VMEM((bpw,), jnp.float32),
            pltpu.SemaphoreType.DMA((bpw // 128,)),
            pltpu.SemaphoreType.DMA,
        ],
    )
    def sc_kernel(dep_hbm, dims_hbm, rots_hbm, lab_hbm,
                  d_out, dims_out, rots_out, alpha_out,
                  lab_v, idxr_v, idxd_v, idx3_v, dep_v, dcol_v, rots_v,
                  rcomp_v, dout_v, alph_v, gsem, osem):
        wid = lax.axis_index("s") * nc + lax.axis_index("c")
        base = wid * bpw
        pltpu.sync_copy(lab_hbm.at[pl.ds(base, bpw)], lab_v)

        iota = lax.iota(jnp.int32, _L)

        def build_idx(g):
            off = g * _L
            c = g // per_chunk
            k = off - c * chunk
            lab = lab_v[pl.ds(off, _L)]
            row = (base + off) + iota
            idxr_v[c, pl.ds(k, _L)] = row * num_classes + lab
            tile = (row >> 7) * 1024 + (row & 127)
            idxd_v[c, pl.ds(k, _L)] = (
                (lab >> 3) * (8 * n) + ((lab & 7) << 7) + tile)
            cd = lab * 3
            for j in range(3):
                cdj = cd + j
                idx3_v[j, c, pl.ds(k, _L)] = (
                    (cdj >> 3) * (8 * n) + ((cdj & 7) << 7) + tile)

        copies = []
        for c in range(nchunks):
            plsc.parallel_loop(
                c * per_chunk, (c + 1) * per_chunk, unroll=2)(build_idx)
            o = c * chunk
            sem = gsem.at[c]
            copies.append(pltpu.async_copy(
                dep_hbm.at[idxd_v.at[c]], dep_v.at[pl.ds(o, chunk)], sem))
            copies.append(pltpu.async_copy(
                rots_hbm.at[idxr_v.at[c]], rots_v.at[pl.ds(o, chunk)], sem))
            for j in range(3):
                copies.append(pltpu.async_copy(
                    dims_hbm.at[idx3_v.at[j].at[c]],
                    dcol_v.at[c].at[j], sem))

        def compute(g):
            off = g * _L
            c = g // per_chunk
            k = off - c * chunk
            dep = dep_v[pl.ds(off, _L)]
            dout_v[pl.ds(off, _L)] = jnp.exp(dep) - 1.0

            rows = off + iota
            r = []
            for j in range(8):
                cj = jnp.full((_L,), j, jnp.int32)
                v = plsc.load_gather(rots_v, [rows, cj])
                rcomp_v[c, j, pl.ds(k, _L)] = v
                r.append(v)
            a1 = _atan_ratio(r[2], r[3]) - _HALF_PI
            a2 = _atan_ratio(r[6], r[7]) + _HALF_PI
            alph_v[pl.ds(off, _L)] = jnp.where(r[1] > r[5], a1, a2)

        for c in range(nchunks):
            for cp in copies[5 * c:5 * c + 5]:
                cp.wait()
            plsc.parallel_loop(
                c * per_chunk, (c + 1) * per_chunk, unroll=2)(compute)

        blk = base // 128
        out_copies = [
            pltpu.async_copy(dout_v, d_out.at[pl.ds(base, bpw)], osem),
            pltpu.async_copy(alph_v, alpha_out.at[pl.ds(base, bpw)], osem),
            pltpu.async_copy(
                dcol_v, dims_out.at[pl.ds(blk, bpw // 128)], osem),
            pltpu.async_copy(
                rcomp_v, rots_out.at[pl.ds(blk, bpw // 128)], osem),
        ]
        for cp in out_copies:
            cp.wait()

    return sc_kernel


def kernel(depths, dims, rots, labels):
    n, num_classes = depths.shape
    dep_t = (depths.T.reshape(num_classes // 8, 8, n // 128, 128)
             .transpose(0, 2, 1, 3).reshape(n * num_classes))
    dims_t = (dims.T.reshape(num_classes * 3 // 8, 8, n // 128, 128)
              .transpose(0, 2, 1, 3).reshape(n * num_classes * 3))
    rots_flat = rots.reshape(n * num_classes, 8)
    lab = labels.astype(jnp.int32)
    d, dims_t4, rots_t8, alphas = _make_sc_kernel(n, num_classes)(
        dep_t, dims_t, rots_flat, lab)
    dims_g = dims_t4[:, :3, :].transpose(0, 2, 1).reshape(n, 3)
    rots_g = rots_t8.transpose(0, 2, 1).reshape(n, 8)
    return d.reshape(n, 1), dims_g, rots_g, alphas


# speedup vs baseline: 4.3912x; 1.0314x over previous
import functools
import math

import jax
import jax.numpy as jnp
from jax import lax
from jax.experimental import pallas as pl
from jax.experimental.pallas import tpu as pltpu
from jax.experimental.pallas import tpu_sc as plsc

_L = 16

_ATAN_C = (
    0.99999612,
    -0.33317369,
    0.19807802,
    -0.13233266,
    0.07962221,
    -0.03360298,
    0.0068114,
)
_HALF_PI = math.pi / 2


def _atan_ratio(num, den):
    an = jnp.abs(num)
    ad = jnp.abs(den)
    t = jnp.minimum(an, ad) / jnp.maximum(an, ad)
    z = t * t
    p = jnp.float32(_ATAN_C[-1])
    for c in _ATAN_C[-2::-1]:
        p = p * z + jnp.float32(c)
    p = p * t
    r = jnp.where(an > ad, _HALF_PI - p, p)
    return jnp.where(num * den < 0.0, -r, r)


def _make_sc_kernel(n, num_classes):
    info = plsc.get_sparse_core_info()
    nc, ns = info.num_cores, info.num_subcores
    nw = nc * ns
    assert n % (nw * _L) == 0
    bpw = n // nw
    groups = bpw // _L
    chunk = 128
    nchunks = bpw // chunk
    per_chunk = chunk // _L

    mesh = plsc.VectorSubcoreMesh(core_axis_name="c", subcore_axis_name="s")

    @functools.partial(
        pl.kernel,
        mesh=mesh,
        compiler_params=pltpu.CompilerParams(
            needs_layout_passes=False, use_tc_tiling_on_sc=False),
        out_type=[
            jax.ShapeDtypeStruct((n,), jnp.float32),
            jax.ShapeDtypeStruct((n // 128, 4, 128), jnp.float32),
            jax.ShapeDtypeStruct((n // 128, 8, 128), jnp.float32),
            jax.ShapeDtypeStruct((n,), jnp.float32),
        ],
        scratch_types=[
            pltpu.VMEM((bpw,), jnp.int32),
            pltpu.VMEM((nchunks, chunk), jnp.int32),
            pltpu.VMEM((3, nchunks, chunk), jnp.int32),
            pltpu.VMEM((num_classes // 8, bpw // 128, 8, 128),
                       jnp.float32),
            pltpu.VMEM((bpw // 128, 4, 128), jnp.float32),
            pltpu.VMEM((bpw, 8), jnp.float32),
            pltpu.VMEM((bpw // 128, 8, 128), jnp.float32),
            pltpu.VMEM((bpw,), jnp.float32),
            pltpu.VMEM((bpw,), jnp.float32),
            pltpu.SemaphoreType.DMA((bpw // 128,)),
            pltpu.SemaphoreType.DMA,
            pltpu.SemaphoreType.DMA,
        ],
    )
    def sc_kernel(dep_hbm, dims_hbm, rots_hbm, lab_hbm,
                  d_out, dims_out, rots_out, alpha_out,
                  lab_v, idxr_v, idx3_v, dep_slab, dcol_v, rots_v,
                  rcomp_v, dout_v, alph_v, gsem, osem, dsem):
        wid = lax.axis_index("s") * nc + lax.axis_index("c")
        base = wid * bpw
        blk = base // 128
        dep_copies = [
            pltpu.async_copy(
                dep_hbm.at[a].at[pl.ds(blk, bpw // 128)],
                dep_slab.at[a], dsem)
            for a in range(num_classes // 8)
        ]
        pltpu.sync_copy(lab_hbm.at[pl.ds(base, bpw)], lab_v)

        iota = lax.iota(jnp.int32, _L)

        def build_idx(g):
            off = g * _L
            c = g // per_chunk
            k = off - c * chunk
            lab = lab_v[pl.ds(off, _L)]
            row = (base + off) + iota
            idxr_v[c, pl.ds(k, _L)] = row * num_classes + lab
            tile = (row >> 7) * 1024 + (row & 127)
            cd = lab * 3
            for j in range(3):
                cdj = cd + j
                idx3_v[j, c, pl.ds(k, _L)] = (
                    (cdj >> 3) * (8 * n) + ((cdj & 7) << 7) + tile)

        copies = []
        for c in range(nchunks):
            plsc.parallel_loop(
                c * per_chunk, (c + 1) * per_chunk, unroll=2)(build_idx)
            o = c * chunk
            sem = gsem.at[c]
            copies.append(pltpu.async_copy(
                rots_hbm.at[idxr_v.at[c]], rots_v.at[pl.ds(o, chunk)], sem))
            for j in range(3):
                copies.append(pltpu.async_copy(
                    dims_hbm.at[idx3_v.at[j].at[c]],
                    dcol_v.at[c].at[j], sem))

        def compute(g):
            off = g * _L
            c = g // per_chunk
            k = off - c * chunk
            lab = lab_v[pl.ds(off, _L)]
            cvec = jnp.zeros((_L,), jnp.int32) + c
            dep = plsc.load_gather(
                dep_slab, [lab >> 3, cvec, lab & 7, k + iota])
            dout_v[pl.ds(off, _L)] = jnp.exp(dep) - 1.0

            rows = off + iota
            r = []
            for j in range(8):
                cj = jnp.full((_L,), j, jnp.int32)
                v = plsc.load_gather(rots_v, [rows, cj])
                rcomp_v[c, j, pl.ds(k, _L)] = v
                r.append(v)
            a1 = _atan_ratio(r[2], r[3]) - _HALF_PI
            a2 = _atan_ratio(r[6], r[7]) + _HALF_PI
            alph_v[pl.ds(off, _L)] = jnp.where(r[1] > r[5], a1, a2)

        for cp in dep_copies:
            cp.wait()
        for c in range(nchunks):
            for cp in copies[4 * c:4 * c + 4]:
                cp.wait()
            plsc.parallel_loop(
                c * per_chunk, (c + 1) * per_chunk, unroll=2)(compute)

        blk = base // 128
        out_copies = [
            pltpu.async_copy(dout_v, d_out.at[pl.ds(base, bpw)], osem),
            pltpu.async_copy(alph_v, alpha_out.at[pl.ds(base, bpw)], osem),
            pltpu.async_copy(
                dcol_v, dims_out.at[pl.ds(blk, bpw // 128)], osem),
            pltpu.async_copy(
                rcomp_v, rots_out.at[pl.ds(blk, bpw // 128)], osem),
        ]
        for cp in out_copies:
            cp.wait()

    return sc_kernel


def kernel(depths, dims, rots, labels):
    n, num_classes = depths.shape
    dep_t = (depths.T.reshape(num_classes // 8, 8, n // 128, 128)
             .transpose(0, 2, 1, 3))
    dims_t = (dims.T.reshape(num_classes * 3 // 8, 8, n // 128, 128)
              .transpose(0, 2, 1, 3).reshape(n * num_classes * 3))
    rots_flat = rots.reshape(n * num_classes, 8)
    lab = labels.astype(jnp.int32)
    d, dims_t4, rots_t8, alphas = _make_sc_kernel(n, num_classes)(
        dep_t, dims_t, rots_flat, lab)
    dims_g = dims_t4[:, :3, :].transpose(0, 2, 1).reshape(n, 3)
    rots_g = rots_t8.transpose(0, 2, 1).reshape(n, 8)
    return d.reshape(n, 1), dims_g, rots_g, alphas
